# baseline (speedup 1.0000x reference)
import functools

import jax
import jax.numpy as jnp
from jax import lax
from jax.experimental import pallas as pl
from jax.experimental.pallas import tpu as pltpu
from jax.experimental.pallas import tpu_sc as plsc

N = 262144
F = 64
RANK = 17
RANK_PAD = 24
NUM_RF = 16

_NC = 2
_NS = 16
_NW = _NC * _NS
_RPW = N // _NW
_CHUNK = 1024
_NBUF = 2


def _yr_body(idx_hbm, table_hbm, x_hbm, out_hbm, idx_v, rows_v, xv, yr_v,
             gsems, xsems):
    wid = lax.axis_index("s") * _NC + lax.axis_index("c")
    base = wid * _RPW
    nch = _RPW // _CHUNK

    pltpu.sync_copy(idx_hbm.at[pl.ds(base, _RPW)], idx_v)

    def start_chunk(ch):
        buf = ch % _NBUF
        gcp = pltpu.make_async_copy(
            table_hbm.at[idx_v.at[pl.ds(ch * _CHUNK, _CHUNK)]],
            rows_v.at[buf], gsems.at[buf])
        gcp.start()
        xcp = pltpu.make_async_copy(
            x_hbm.at[pl.ds(base + ch * _CHUNK, _CHUNK), pl.ds(0, NUM_RF)],
            xv.at[buf], xsems.at[buf])
        xcp.start()
        return gcp, xcp

    iota = lax.iota(jnp.int32, 16)
    cols_b0 = jnp.full((16,), NUM_RF, jnp.int32)

    def compute_chunk(ch):
        buf = ch % _NBUF
        rows = rows_v.at[buf]
        xb = xv.at[buf]

        def group(i, _):
            r = i * 16 + iota
            acc = plsc.load_gather(rows, [r, cols_b0])
            for k in range(NUM_RF):
                bk = plsc.load_gather(rows, [r, jnp.full((16,), k, jnp.int32)])
                xk = plsc.load_gather(xb, [r, jnp.full((16,), k, jnp.int32)])
                acc = acc + xk * bk
            plsc.store_scatter(yr_v, [ch * _CHUNK + r], acc)
            return _

        lax.fori_loop(0, _CHUNK // 16, group, 0, unroll=False)

    copies = [None] * _NBUF
    copies[0] = start_chunk(0)
    for ch in range(nch):
        if ch + 1 < nch:
            copies[(ch + 1) % _NBUF] = start_chunk(ch + 1)
        gcp, xcp = copies[ch % _NBUF]
        gcp.wait()
        xcp.wait()
        compute_chunk(ch)
    pltpu.sync_copy(yr_v, out_hbm.at[pl.ds(base, _RPW)])


@jax.jit
def _sc_yr(group_ids, table, X):
    mesh = plsc.VectorSubcoreMesh(core_axis_name="c", subcore_axis_name="s")
    return pl.kernel(
        _yr_body,
        out_type=jax.ShapeDtypeStruct((N,), jnp.float32),
        mesh=mesh,
        compiler_params=pltpu.CompilerParams(use_tc_tiling_on_sc=False,
                                             needs_layout_passes=False),
        scratch_types=[
            pltpu.VMEM((_RPW,), jnp.int32),
            pltpu.VMEM((_NBUF, _CHUNK, RANK_PAD), jnp.float32),
            pltpu.VMEM((_NBUF, _CHUNK, NUM_RF), jnp.float32),
            pltpu.VMEM((_RPW,), jnp.float32),
            pltpu.SemaphoreType.DMA((_NBUF,)),
            pltpu.SemaphoreType.DMA((_NBUF,)),
        ],
    )(group_ids, table, X)


_BLK = 16384


def _dense_body(x_ref, yr_ref, rhs_ref, o_ref):
    x = x_ref[...]
    yr = yr_ref[...]
    ones = jnp.ones_like(yr)
    cat = jnp.concatenate([x, yr, ones], axis=1)
    o_ref[...] = jax.lax.dot_general(
        cat, rhs_ref[...], (((1,), (0,)), ((), ())),
        preferred_element_type=jnp.float32)


@jax.jit
def _tc_dense(X, yr, rhs):
    grid = (N // _BLK,)
    return pl.pallas_call(
        _dense_body,
        out_shape=jax.ShapeDtypeStruct((N, 1), jnp.float32),
        grid=grid,
        in_specs=[
            pl.BlockSpec((_BLK, F), lambda i: (i, 0)),
            pl.BlockSpec((_BLK, 1), lambda i: (i, 0)),
            pl.BlockSpec((F + 2, 1), lambda i: (0, 0)),
        ],
        out_specs=pl.BlockSpec((_BLK, 1), lambda i: (i, 0)),
    )(X, yr, rhs)


def kernel(X, group_ids, res_per_gf, W, b):
    G = res_per_gf.shape[0]
    table = jnp.concatenate(
        [res_per_gf[:, 1:], res_per_gf[:, :1],
         jnp.zeros((G, RANK_PAD - RANK), jnp.float32)],
        axis=1)
    rhs = jnp.concatenate(
        [W.T,
         jnp.ones((1, 1), jnp.float32),
         b.reshape(1, 1)],
        axis=0)
    yr = _sc_yr(group_ids.astype(jnp.int32), table, X)
    return _tc_dense(X, yr.reshape(N, 1), rhs).reshape(N)


# speedup vs baseline: 1.1250x; 1.1250x over previous
import jax
import jax.numpy as jnp
from jax import lax
from jax.experimental import pallas as pl
from jax.experimental.pallas import tpu as pltpu
from jax.experimental.pallas import tpu_sc as plsc

N = 262144
F = 64
RANK = 17
RANK_PAD = 24
NUM_RF = 16

_NC = 2
_NS = 16
_NW = _NC * _NS
_RPW = N // _NW
_CHUNK = 512
_NBUF = 2
_GRP = _CHUNK // 16


def _body(idx_hbm, table_hbm, xflat_hbm, wb_hbm, out_hbm,
          idx_v, rows_v, xv, yb, wv, gsems, xsems):
    wid = lax.axis_index("s") * _NC + lax.axis_index("c")
    base = wid * _RPW
    nch = _RPW // _CHUNK

    pltpu.sync_copy(idx_hbm.at[pl.ds(base, _RPW)], idx_v)
    pltpu.sync_copy(wb_hbm, wv)
    wvecs = [wv[pl.ds(16 * j, 16)] for j in range(5)]
    ws = [wvecs[k // 16][k % 16] for k in range(F)]
    bias = wvecs[4][0]

    iota = lax.iota(jnp.int32, 16)
    iota64 = iota * F
    cols = [jnp.full((16,), k, jnp.int32) for k in range(RANK)]

    def copies(ch, buf):
        gcp = pltpu.make_async_copy(
            table_hbm.at[idx_v.at[pl.ds(ch * _CHUNK, _CHUNK)]],
            rows_v.at[buf], gsems.at[buf])
        xcp = pltpu.make_async_copy(
            xflat_hbm.at[pl.ds((base + ch * _CHUNK) * F, _CHUNK * F)],
            xv.at[buf], xsems.at[buf])
        return gcp, xcp

    def start_chunk(ch, buf):
        gcp, xcp = copies(ch, buf)
        gcp.start()
        xcp.start()

    def compute_chunk(ch, buf):
        rows = rows_v.at[buf]
        xb = xv.at[buf]

        def group(i, carry):
            r = i * 16 + iota
            xidx = i * (16 * F) + iota64
            acc_r = plsc.load_gather(rows, [r, cols[NUM_RF]])
            acc_f = jnp.full((16,), 0.0, jnp.float32)
            for k in range(F):
                xk = plsc.load_gather(xb, [xidx])
                xidx = xidx + 1
                acc_f = acc_f + xk * ws[k]
                if k < NUM_RF:
                    bk = plsc.load_gather(rows, [r, cols[k]])
                    acc_r = acc_r + xk * bk
            plsc.store_scatter(yb, [r], acc_f + acc_r + bias)
            return carry

        lax.fori_loop(0, _GRP, group, 0)
        pltpu.sync_copy(yb, out_hbm.at[pl.ds(base + ch * _CHUNK, _CHUNK)])

    for b in range(_NBUF):
        start_chunk(b, b)

    def ring(g, carry):
        for b in range(_NBUF):
            ch = g * _NBUF + b
            gcp, xcp = copies(ch, b)
            gcp.wait()
            xcp.wait()
            compute_chunk(ch, b)

            @pl.when(ch + _NBUF < nch)
            def _():
                start_chunk(ch + _NBUF, b)
        return carry

    lax.fori_loop(0, nch // _NBUF, ring, 0)


@jax.jit
def _sc_all(group_ids, table, xflat, wb):
    mesh = plsc.VectorSubcoreMesh(core_axis_name="c", subcore_axis_name="s")
    return pl.kernel(
        _body,
        out_type=jax.ShapeDtypeStruct((N,), jnp.float32),
        mesh=mesh,
        compiler_params=pltpu.CompilerParams(use_tc_tiling_on_sc=False,
                                             needs_layout_passes=False),
        scratch_types=[
            pltpu.VMEM((_RPW,), jnp.int32),
            pltpu.VMEM((_NBUF, _CHUNK, RANK_PAD), jnp.float32),
            pltpu.VMEM((_NBUF, _CHUNK * F), jnp.float32),
            pltpu.VMEM((_CHUNK,), jnp.float32),
            pltpu.VMEM((F + 16,), jnp.float32),
            pltpu.SemaphoreType.DMA((_NBUF,)),
            pltpu.SemaphoreType.DMA((_NBUF,)),
        ],
    )(group_ids, table, xflat, wb)


def kernel(X, group_ids, res_per_gf, W, b):
    G = res_per_gf.shape[0]
    table = jnp.concatenate(
        [res_per_gf[:, 1:], res_per_gf[:, :1],
         jnp.zeros((G, RANK_PAD - RANK), jnp.float32)],
        axis=1)
    wb = jnp.concatenate(
        [W.reshape(F), b.reshape(1), jnp.zeros((15,), jnp.float32)])
    return _sc_all(group_ids.astype(jnp.int32), table, X.reshape(N * F), wb)


# speedup vs baseline: 1.8947x; 1.6841x over previous
import jax
import jax.numpy as jnp
from jax import lax
from jax.experimental import pallas as pl
from jax.experimental.pallas import tpu as pltpu
from jax.experimental.pallas import tpu_sc as plsc

N = 262144
F = 64
RANK = 17
RANK_PAD = 24
NUM_RF = 16

_NC = 2
_NS = 16
_NW = _NC * _NS
_RPW = N // _NW
_CHUNK = 512
_NBUF = 2
_GRP = _CHUNK // 16


def _body(idx_hbm, table_hbm, xflat_hbm, wskew_hbm, bias_hbm, out_hbm,
          idx_v, rows_v, xv, yb, wsk_v, bias_v, gsems, xsems):
    wid = lax.axis_index("s") * _NC + lax.axis_index("c")
    base = wid * _RPW
    nch = _RPW // _CHUNK

    pltpu.sync_copy(idx_hbm.at[pl.ds(base, _RPW)], idx_v)
    pltpu.sync_copy(wskew_hbm, wsk_v)
    pltpu.sync_copy(bias_hbm, bias_v)
    bias = bias_v[0:16][0]

    iota = lax.iota(jnp.int32, 16)
    iota64 = iota * F
    rot5 = iota * 5
    col_b0 = jnp.full((16,), NUM_RF, jnp.int32)

    def copies(ch, buf):
        gcp = pltpu.make_async_copy(
            table_hbm.at[idx_v.at[pl.ds(ch * _CHUNK, _CHUNK)]],
            rows_v.at[buf], gsems.at[buf])
        xcp = pltpu.make_async_copy(
            xflat_hbm.at[pl.ds((base + ch * _CHUNK) * F, _CHUNK * F)],
            xv.at[buf], xsems.at[buf])
        return gcp, xcp

    def start_chunk(ch, buf):
        gcp, xcp = copies(ch, buf)
        gcp.start()
        xcp.start()

    def compute_chunk(ch, buf):
        rows = rows_v.at[buf]
        xb = xv.at[buf]

        def group(i, carry):
            r = i * 16 + iota
            rb64 = i * (16 * F) + iota64
            acc_r = plsc.load_gather(rows, [r, col_b0])
            acc_f = jnp.full((16,), 0.0, jnp.float32)
            for k in range(F):
                c = jnp.bitwise_and(rot5 + k, F - 1)
                xk = plsc.load_gather(xb, [rb64 + c])
                wk = wsk_v[k]
                acc_f = acc_f + xk * wk
            for k in range(NUM_RF):
                c = jnp.bitwise_and(rot5 + k, NUM_RF - 1)
                xk = plsc.load_gather(xb, [rb64 + c])
                bk = plsc.load_gather(rows, [r, c])
                acc_r = acc_r + xk * bk
            plsc.store_scatter(yb, [r], acc_f + acc_r + bias)
            return carry

        lax.fori_loop(0, _GRP, group, 0)
        pltpu.sync_copy(yb, out_hbm.at[pl.ds(base + ch * _CHUNK, _CHUNK)])

    for b in range(_NBUF):
        start_chunk(b, b)

    def ring(g, carry):
        for b in range(_NBUF):
            ch = g * _NBUF + b
            gcp, xcp = copies(ch, b)
            gcp.wait()
            xcp.wait()
            compute_chunk(ch, b)

            @pl.when(ch + _NBUF < nch)
            def _():
                start_chunk(ch + _NBUF, b)
        return carry

    lax.fori_loop(0, nch // _NBUF, ring, 0)


@jax.jit
def _sc_all(group_ids, table, xflat, wskew, bias):
    mesh = plsc.VectorSubcoreMesh(core_axis_name="c", subcore_axis_name="s")
    return pl.kernel(
        _body,
        out_type=jax.ShapeDtypeStruct((N,), jnp.float32),
        mesh=mesh,
        compiler_params=pltpu.CompilerParams(use_tc_tiling_on_sc=False,
                                             needs_layout_passes=False),
        scratch_types=[
            pltpu.VMEM((_RPW,), jnp.int32),
            pltpu.VMEM((_NBUF, _CHUNK, RANK_PAD), jnp.float32),
            pltpu.VMEM((_NBUF, _CHUNK * F), jnp.float32),
            pltpu.VMEM((_CHUNK,), jnp.float32),
            pltpu.VMEM((F, 16), jnp.float32),
            pltpu.VMEM((16,), jnp.float32),
            pltpu.SemaphoreType.DMA((_NBUF,)),
            pltpu.SemaphoreType.DMA((_NBUF,)),
        ],
    )(group_ids, table, xflat, wskew, bias)


def kernel(X, group_ids, res_per_gf, W, b):
    G = res_per_gf.shape[0]
    table = jnp.concatenate(
        [res_per_gf[:, 1:], res_per_gf[:, :1],
         jnp.zeros((G, RANK_PAD - RANK), jnp.float32)],
        axis=1)
    w0 = W.reshape(F)
    skew_idx = (jnp.arange(F)[:, None] + 5 * jnp.arange(16)[None, :]) % F
    wskew = w0[skew_idx]
    bias = jnp.concatenate([b.reshape(1), jnp.zeros((15,), jnp.float32)])
    return _sc_all(group_ids.astype(jnp.int32), table, X.reshape(N * F),
                   wskew, bias)
